# baseline (speedup 1.0000x reference)
import jax
import jax.numpy as jnp
from jax import lax
from jax.experimental import pallas as pl
from jax.experimental.pallas import tpu as pltpu
from jax.experimental.pallas import tpu_sc as plsc

DIM = 64
PAD_IDX = 0
SCALE = 8.0
NC = 2
NS = 16
L = 16
NW = NC * NS
BW = 128
NBUF = 4
SEQ = 200
BANDS = SEQ // 8


def _body(table_hbm, idx_hbm, out_hbm, idx_v, idx2_v, rows_v, out_v,
          sem_g0, sem_g1, sem_g2, sem_g3, sem_w0, sem_w1):
    wid = lax.axis_index("c") * NS + lax.axis_index("s")
    b0 = wid * BW
    sem_g = [sem_g0, sem_g1, sem_g2, sem_g3]
    sem_w = [sem_w0, sem_w1]

    pltpu.sync_copy(idx_hbm.at[:, wid], idx_v)

    iota = lax.iota(jnp.int32, L)

    def issue_gather(s, j):
        for g in range(BW // L):
            sl = pl.ds(g * L, L)
            idx2_v.at[j][sl] = lax.shift_right_logical(
                idx_v[s >> 3, s & 7, sl], 1)
        pltpu.async_copy(table_hbm.at[idx2_v.at[j]], rows_v.at[j], sem_g[j])

    def wait_gather(j):
        pltpu.make_async_copy(table_hbm.at[pl.ds(0, BW)], rows_v.at[j],
                              sem_g[j]).wait()

    def wait_writeback(s, oj):
        pltpu.make_async_copy(out_v.at[oj],
                              out_hbm.at[s].at[:, pl.ds(b0, BW)],
                              sem_w[oj]).wait()

    def compute(s, j, oj):
        rows = rows_v.at[j]
        dst = out_v.at[oj]

        for g in range(BW // L):
            tok = idx_v[s >> 3, s & 7, pl.ds(g * L, L)]
            half = (tok & 1) << 6
            factor = jnp.where(tok == PAD_IDX, 0.0, SCALE)
            rowv = iota + g * L

            @plsc.parallel_loop(0, DIM, unroll=16)
            def _(d):
                colv = half | d
                v = plsc.load_gather(rows, [rowv, colv])
                dst.at[d][pl.ds(g * L, L)] = v * factor

    for j in range(NBUF - 1):
        issue_gather(j, j)

    @pl.loop(0, SEQ // NBUF)
    def _(sblk):
        for j in range(NBUF):
            s = sblk * NBUF + j
            oj = j & 1

            @pl.when(s < SEQ - (NBUF - 1))
            def _():
                issue_gather(s + NBUF - 1, (j + NBUF - 1) % NBUF)

            wait_gather(j)

            @pl.when(s >= 2)
            def _():
                wait_writeback(s - 2, oj)

            compute(s, j, oj)
            pltpu.async_copy(out_v.at[oj],
                             out_hbm.at[s].at[:, pl.ds(b0, BW)],
                             sem_w[oj])

    wait_writeback(SEQ - 2, 0)
    wait_writeback(SEQ - 1, 1)


@jax.jit
def _embed(idx4, table2):
    batch = idx4.shape[1] * idx4.shape[3]
    mesh = plsc.VectorSubcoreMesh(core_axis_name="c", subcore_axis_name="s")
    cp = pltpu.CompilerParams(needs_layout_passes=False,
                              use_tc_tiling_on_sc=True)
    kern = pl.kernel(
        _body,
        out_type=jax.ShapeDtypeStruct((SEQ, DIM, batch), jnp.float32),
        mesh=mesh,
        scratch_types=[
            pltpu.VMEM((BANDS, 8, BW), jnp.int32),
            pltpu.VMEM((NBUF, BW), jnp.int32),
            pltpu.VMEM((NBUF, BW, 2 * DIM), jnp.float32),
            pltpu.VMEM((2, DIM, BW), jnp.float32),
            pltpu.SemaphoreType.DMA,
            pltpu.SemaphoreType.DMA,
            pltpu.SemaphoreType.DMA,
            pltpu.SemaphoreType.DMA,
            pltpu.SemaphoreType.DMA,
            pltpu.SemaphoreType.DMA,
        ],
        compiler_params=cp,
    )
    return kern(table2, idx4)


def kernel(token_tensor, table):
    tt = token_tensor.astype(jnp.int32)
    batch, seq = tt.shape
    idx4 = jnp.transpose(
        jnp.transpose(tt).reshape(seq // 8, 8, batch // BW, BW),
        (0, 2, 1, 3))
    half_rows = table.shape[0] // 2
    table2 = jnp.transpose(
        jnp.transpose(table).reshape(DIM, half_rows, 2),
        (1, 2, 0)).reshape(half_rows, 2 * DIM)
    out = _embed(idx4, table2)
    return jnp.transpose(out, (2, 0, 1))


# speedup vs baseline: 1.5070x; 1.5070x over previous
import jax
import jax.numpy as jnp
from jax import lax
from jax.experimental import pallas as pl
from jax.experimental.pallas import tpu as pltpu
from jax.experimental.pallas import tpu_sc as plsc

DIM = 64
PAD_IDX = 0
SCALE = 8.0
NC = 2
NS = 16
L = 16
NW = NC * NS
BW = 128
NBUF = 4
SEQ = 200
BANDS = SEQ // 8


def _body(table_hbm, idx_hbm, out_hbm, idx_v, idx2_v, rows_v, out_v,
          sem_g0, sem_g1, sem_g2, sem_g3, sem_w0, sem_w1):
    wid = lax.axis_index("c") * NS + lax.axis_index("s")
    b0 = wid * BW
    sem_g = [sem_g0, sem_g1, sem_g2, sem_g3]
    sem_w = [sem_w0, sem_w1]

    pltpu.sync_copy(idx_hbm.at[:, wid], idx_v)

    iota = lax.iota(jnp.int32, L)

    def issue_gather(s, j):
        for g in range(BW // L):
            sl = pl.ds(g * L, L)
            tok = idx_v[s >> 3, s & 7, sl]
            idx2_v.at[j][sl] = tok & (SPLIT - 1)
        pltpu.async_copy(table_hbm.at[idx2_v.at[j]], rows_v.at[j], sem_g[j])

    def wait_gather(j):
        pltpu.make_async_copy(table_hbm.at[pl.ds(0, BW)], rows_v.at[j],
                              sem_g[j]).wait()

    def wait_writeback(s, oj):
        pltpu.make_async_copy(out_v.at[oj],
                              out_hbm.at[s].at[:, pl.ds(b0, BW)],
                              sem_w[oj]).wait()

    def compute(s, j, oj):
        rows = rows_v.at[j]
        dst = out_v.at[oj]

        for g in range(BW // L):
            tok = idx_v[s >> 3, s & 7, pl.ds(g * L, L)]
            half = (tok >> 19) << 6
            rowv = iota + g * L

            @plsc.parallel_loop(0, DIM, unroll=16)
            def _(d):
                colv = half | d
                v = plsc.load_gather(rows, [rowv, colv])
                dst.at[d][pl.ds(g * L, L)] = v

    for j in range(NBUF - 1):
        issue_gather(j, j)

    @pl.loop(0, SEQ // NBUF)
    def _(sblk):
        for j in range(NBUF):
            s = sblk * NBUF + j
            oj = j & 1

            @pl.when(s < SEQ - (NBUF - 1))
            def _():
                issue_gather(s + NBUF - 1, (j + NBUF - 1) % NBUF)

            wait_gather(j)

            @pl.when(s >= 2)
            def _():
                wait_writeback(s - 2, oj)

            compute(s, j, oj)
            pltpu.async_copy(out_v.at[oj],
                             out_hbm.at[s].at[:, pl.ds(b0, BW)],
                             sem_w[oj])

    wait_writeback(SEQ - 2, 0)
    wait_writeback(SEQ - 1, 1)


PREP_BK = 4096
SPLIT = 524288


def _prep_body(lo_ref, hi_ref, out_ref):
    out_ref[:, 0:DIM] = jnp.transpose(lo_ref[...] * SCALE)
    out_ref[:, DIM:2 * DIM] = jnp.transpose(hi_ref[...] * SCALE)

    @pl.when(pl.program_id(0) == 0)
    def _():
        out_ref[0:1, 0:DIM] = jnp.zeros((1, DIM), jnp.float32)


def _prep(table_t):
    grid = SPLIT // PREP_BK
    last = (table_t.shape[1] + PREP_BK - 1) // PREP_BK - 1
    return pl.pallas_call(
        _prep_body,
        grid=(grid,),
        in_specs=[
            pl.BlockSpec((DIM, PREP_BK), lambda i: (0, i)),
            pl.BlockSpec((DIM, PREP_BK),
                         lambda i: (0, jnp.minimum(i + grid, last))),
        ],
        out_specs=pl.BlockSpec((PREP_BK, 2 * DIM), lambda i: (i, 0)),
        out_shape=jax.ShapeDtypeStruct((SPLIT, 2 * DIM), jnp.float32),
    )(table_t, table_t)


@jax.jit
def _embed(idx4, table2):
    batch = idx4.shape[1] * idx4.shape[3]
    mesh = plsc.VectorSubcoreMesh(core_axis_name="c", subcore_axis_name="s")
    cp = pltpu.CompilerParams(needs_layout_passes=False,
                              use_tc_tiling_on_sc=True)
    kern = pl.kernel(
        _body,
        out_type=jax.ShapeDtypeStruct((SEQ, DIM, batch), jnp.float32),
        mesh=mesh,
        scratch_types=[
            pltpu.VMEM((BANDS, 8, BW), jnp.int32),
            pltpu.VMEM((NBUF, BW), jnp.int32),
            pltpu.VMEM((NBUF, BW, 2 * DIM), jnp.float32),
            pltpu.VMEM((2, DIM, BW), jnp.float32),
            pltpu.SemaphoreType.DMA,
            pltpu.SemaphoreType.DMA,
            pltpu.SemaphoreType.DMA,
            pltpu.SemaphoreType.DMA,
            pltpu.SemaphoreType.DMA,
            pltpu.SemaphoreType.DMA,
        ],
        compiler_params=cp,
    )
    return kern(table2, idx4)


@jax.jit
def kernel(token_tensor, table):
    tt = token_tensor.astype(jnp.int32)
    batch, seq = tt.shape
    idx4 = jnp.transpose(
        jnp.transpose(tt).reshape(seq // 8, 8, batch // BW, BW),
        (0, 2, 1, 3))
    table2 = _prep(jnp.transpose(table))
    out = _embed(idx4, table2)
    return jnp.transpose(out, (2, 0, 1))


# speedup vs baseline: 1.5128x; 1.0038x over previous
import jax
import jax.numpy as jnp
from jax import lax
from jax.experimental import pallas as pl
from jax.experimental.pallas import tpu as pltpu
from jax.experimental.pallas import tpu_sc as plsc

DIM = 64
PAD_IDX = 0
SCALE = 8.0
NC = 2
NS = 16
L = 16
NW = NC * NS
BW = 128
NBUF = 4
SEQ = 200
BANDS = SEQ // 8


def _body(table_hbm, idx_hbm, out_hbm, idx_v, idx2_v, rows_v, out_v,
          sem_g0, sem_g1, sem_g2, sem_g3, sem_w0, sem_w1):
    wid = lax.axis_index("c") * NS + lax.axis_index("s")
    b0 = wid * BW
    sem_g = [sem_g0, sem_g1, sem_g2, sem_g3]
    sem_w = [sem_w0, sem_w1]

    pltpu.sync_copy(idx_hbm.at[:, wid], idx_v)

    iota = lax.iota(jnp.int32, L)

    def issue_gather(s, j):
        for g in range(BW // L):
            sl = pl.ds(g * L, L)
            tok = idx_v[s >> 3, s & 7, sl]
            idx2_v.at[j][sl] = tok & (SPLIT - 1)
        pltpu.async_copy(table_hbm.at[idx2_v.at[j]], rows_v.at[j], sem_g[j])

    def wait_gather(j):
        pltpu.make_async_copy(table_hbm.at[pl.ds(0, BW)], rows_v.at[j],
                              sem_g[j]).wait()

    def wait_writeback(s, oj):
        pltpu.make_async_copy(out_v.at[oj],
                              out_hbm.at[s].at[:, pl.ds(b0, BW)],
                              sem_w[oj]).wait()

    def compute(s, j, oj):
        rows = rows_v.at[j]
        dst = out_v.at[oj]

        for g in range(BW // L):
            tok = idx_v[s >> 3, s & 7, pl.ds(g * L, L)]
            half = (tok >> 19) << 6
            rowv = iota + g * L

            @plsc.parallel_loop(0, DIM, unroll=16)
            def _(d):
                colv = half | d
                v = plsc.load_gather(rows, [rowv, colv])
                dst.at[d][pl.ds(g * L, L)] = v

    for j in range(NBUF - 1):
        issue_gather(j, j)

    @pl.loop(0, SEQ // NBUF)
    def _(sblk):
        for j in range(NBUF):
            s = sblk * NBUF + j
            oj = j & 1

            @pl.when(s < SEQ - (NBUF - 1))
            def _():
                issue_gather(s + NBUF - 1, (j + NBUF - 1) % NBUF)

            wait_gather(j)

            @pl.when(s >= 2)
            def _():
                wait_writeback(s - 2, oj)

            compute(s, j, oj)
            pltpu.async_copy(out_v.at[oj],
                             out_hbm.at[s].at[:, pl.ds(b0, BW)],
                             sem_w[oj])

    wait_writeback(SEQ - 2, 0)
    wait_writeback(SEQ - 1, 1)


PREP_BK = 4096
SPLIT = 524288


def _prep_body(lo_ref, hi_ref, out_ref):
    out_ref[:, 0:DIM] = jnp.transpose(lo_ref[...] * SCALE)
    out_ref[:, DIM:2 * DIM] = jnp.transpose(hi_ref[...] * SCALE)

    @pl.when(pl.program_id(0) == 0)
    def _():
        out_ref[0:1, 0:DIM] = jnp.zeros((1, DIM), jnp.float32)


def _prep(table_t):
    grid = SPLIT // PREP_BK
    last = (table_t.shape[1] + PREP_BK - 1) // PREP_BK - 1
    return pl.pallas_call(
        _prep_body,
        grid=(grid,),
        in_specs=[
            pl.BlockSpec((DIM, PREP_BK), lambda i: (0, i)),
            pl.BlockSpec((DIM, PREP_BK),
                         lambda i: (0, jnp.minimum(i + grid, last))),
        ],
        out_specs=pl.BlockSpec((PREP_BK, 2 * DIM), lambda i: (i, 0)),
        out_shape=jax.ShapeDtypeStruct((SPLIT, 2 * DIM), jnp.float32),
        compiler_params=pltpu.CompilerParams(
            dimension_semantics=("parallel",)),
    )(table_t, table_t)


@jax.jit
def _embed(idx4, table2):
    batch = idx4.shape[1] * idx4.shape[3]
    mesh = plsc.VectorSubcoreMesh(core_axis_name="c", subcore_axis_name="s")
    cp = pltpu.CompilerParams(needs_layout_passes=False,
                              use_tc_tiling_on_sc=True)
    kern = pl.kernel(
        _body,
        out_type=jax.ShapeDtypeStruct((SEQ, DIM, batch), jnp.float32),
        mesh=mesh,
        scratch_types=[
            pltpu.VMEM((BANDS, 8, BW), jnp.int32),
            pltpu.VMEM((NBUF, BW), jnp.int32),
            pltpu.VMEM((NBUF, BW, 2 * DIM), jnp.float32),
            pltpu.VMEM((2, DIM, BW), jnp.float32),
            pltpu.SemaphoreType.DMA,
            pltpu.SemaphoreType.DMA,
            pltpu.SemaphoreType.DMA,
            pltpu.SemaphoreType.DMA,
            pltpu.SemaphoreType.DMA,
            pltpu.SemaphoreType.DMA,
        ],
        compiler_params=cp,
    )
    return kern(table2, idx4)


@jax.jit
def kernel(token_tensor, table):
    tt = token_tensor.astype(jnp.int32)
    batch, seq = tt.shape
    idx4 = jnp.transpose(
        jnp.transpose(tt).reshape(seq // 8, 8, batch // BW, BW),
        (0, 2, 1, 3))
    table2 = _prep(jnp.transpose(table))
    out = _embed(idx4, table2)
    return jnp.transpose(out, (2, 0, 1))


# speedup vs baseline: 1.5666x; 1.0356x over previous
import jax
import jax.numpy as jnp
from jax import lax
from jax.experimental import pallas as pl
from jax.experimental.pallas import tpu as pltpu
from jax.experimental.pallas import tpu_sc as plsc

DIM = 64
PAD_IDX = 0
SCALE = 8.0
NC = 2
NS = 16
L = 16
NW = NC * NS
BW = 128
NBUF = 4
SEQ = 200
BANDS = SEQ // 8


def _body(table_hbm, idx_hbm, out_hbm, idx_v, idx2_v, rows_v, out_v,
          sem_g0, sem_g1, sem_g2, sem_g3, sem_w0, sem_w1):
    wid = lax.axis_index("c") * NS + lax.axis_index("s")
    b0 = wid * BW
    sem_g = [sem_g0, sem_g1, sem_g2, sem_g3]
    sem_w = [sem_w0, sem_w1]

    pltpu.sync_copy(idx_hbm.at[:, wid], idx_v)

    iota = lax.iota(jnp.int32, L)

    def issue_gather(s, j):
        for g in range(BW // L):
            sl = pl.ds(g * L, L)
            tok = idx_v[s >> 3, s & 7, sl]
            idx2_v.at[j][sl] = tok & (SPLIT - 1)
        pltpu.async_copy(table_hbm.at[idx2_v.at[j]], rows_v.at[j], sem_g[j])

    def wait_gather(j):
        pltpu.make_async_copy(table_hbm.at[pl.ds(0, BW)], rows_v.at[j],
                              sem_g[j]).wait()

    def wait_writeback(s, oj):
        pltpu.make_async_copy(out_v.at[oj],
                              out_hbm.at[s].at[:, pl.ds(b0, BW)],
                              sem_w[oj]).wait()

    def compute(s, j, oj):
        rows = rows_v.at[j]
        dst = out_v.at[oj]

        for g in range(BW // L):
            tok = idx_v[s >> 3, s & 7, pl.ds(g * L, L)]
            half = (tok >> 19) << 6
            rowv = iota + g * L

            @plsc.parallel_loop(0, DIM, unroll=16)
            def _(d):
                colv = half | d
                v = plsc.load_gather(rows, [rowv, colv])
                dst.at[d][pl.ds(g * L, L)] = v

    for j in range(NBUF - 1):
        issue_gather(j, j)

    @pl.loop(0, SEQ // NBUF)
    def _(sblk):
        for j in range(NBUF):
            s = sblk * NBUF + j
            oj = j & 1

            @pl.when(s < SEQ - (NBUF - 1))
            def _():
                issue_gather(s + NBUF - 1, (j + NBUF - 1) % NBUF)

            wait_gather(j)

            @pl.when(s >= 2)
            def _():
                wait_writeback(s - 2, oj)

            compute(s, j, oj)
            pltpu.async_copy(out_v.at[oj],
                             out_hbm.at[s].at[:, pl.ds(b0, BW)],
                             sem_w[oj])

    wait_writeback(SEQ - 2, 0)
    wait_writeback(SEQ - 1, 1)


PREP_BK = 8192
SPLIT = 524288


def _prep_body(lo_ref, hi_ref, out_ref):
    out_ref[:, 0:DIM] = jnp.transpose(lo_ref[...] * SCALE)
    out_ref[:, DIM:2 * DIM] = jnp.transpose(hi_ref[...] * SCALE)

    @pl.when(pl.program_id(0) == 0)
    def _():
        out_ref[0:1, 0:DIM] = jnp.zeros((1, DIM), jnp.float32)


def _prep(table_t):
    grid = SPLIT // PREP_BK
    last = (table_t.shape[1] + PREP_BK - 1) // PREP_BK - 1
    return pl.pallas_call(
        _prep_body,
        grid=(grid,),
        in_specs=[
            pl.BlockSpec((DIM, PREP_BK), lambda i: (0, i)),
            pl.BlockSpec((DIM, PREP_BK),
                         lambda i: (0, jnp.minimum(i + grid, last))),
        ],
        out_specs=pl.BlockSpec((PREP_BK, 2 * DIM), lambda i: (i, 0)),
        out_shape=jax.ShapeDtypeStruct((SPLIT, 2 * DIM), jnp.float32),
        compiler_params=pltpu.CompilerParams(
            dimension_semantics=("parallel",)),
    )(table_t, table_t)


@jax.jit
def _embed(idx4, table2):
    batch = idx4.shape[1] * idx4.shape[3]
    mesh = plsc.VectorSubcoreMesh(core_axis_name="c", subcore_axis_name="s")
    cp = pltpu.CompilerParams(needs_layout_passes=False,
                              use_tc_tiling_on_sc=True)
    kern = pl.kernel(
        _body,
        out_type=jax.ShapeDtypeStruct((SEQ, DIM, batch), jnp.float32),
        mesh=mesh,
        scratch_types=[
            pltpu.VMEM((BANDS, 8, BW), jnp.int32),
            pltpu.VMEM((NBUF, BW), jnp.int32),
            pltpu.VMEM((NBUF, BW, 2 * DIM), jnp.float32),
            pltpu.VMEM((2, DIM, BW), jnp.float32),
            pltpu.SemaphoreType.DMA,
            pltpu.SemaphoreType.DMA,
            pltpu.SemaphoreType.DMA,
            pltpu.SemaphoreType.DMA,
            pltpu.SemaphoreType.DMA,
            pltpu.SemaphoreType.DMA,
        ],
        compiler_params=cp,
    )
    return kern(table2, idx4)


@jax.jit
def kernel(token_tensor, table):
    tt = token_tensor.astype(jnp.int32)
    batch, seq = tt.shape
    idx4 = jnp.transpose(
        jnp.transpose(tt).reshape(seq // 8, 8, batch // BW, BW),
        (0, 2, 1, 3))
    table2 = _prep(jnp.transpose(table))
    out = _embed(idx4, table2)
    return jnp.transpose(out, (2, 0, 1))


# speedup vs baseline: 1.5915x; 1.0159x over previous
import jax
import jax.numpy as jnp
from jax import lax
from jax.experimental import pallas as pl
from jax.experimental.pallas import tpu as pltpu
from jax.experimental.pallas import tpu_sc as plsc

DIM = 64
PAD_IDX = 0
SCALE = 8.0
NC = 2
NS = 16
L = 16
NW = NC * NS
BW = 128
NBUF = 4
SEQ = 200
BANDS = SEQ // 8


def _body(table_hbm, idx_hbm, out_hbm, idx_v, idx2_v, rows_v, out_v,
          sem_g0, sem_g1, sem_g2, sem_g3, sem_w0, sem_w1):
    wid = lax.axis_index("c") * NS + lax.axis_index("s")
    b0 = wid * BW
    sem_g = [sem_g0, sem_g1, sem_g2, sem_g3]
    sem_w = [sem_w0, sem_w1]

    pltpu.sync_copy(idx_hbm.at[:, wid], idx_v)

    iota = lax.iota(jnp.int32, L)

    def issue_gather(s, j):
        for g in range(BW // L):
            sl = pl.ds(g * L, L)
            tok = idx_v[s >> 3, s & 7, sl]
            idx2_v.at[j][sl] = tok & (SPLIT - 1)
        pltpu.async_copy(table_hbm.at[idx2_v.at[j]], rows_v.at[j], sem_g[j])

    def wait_gather(j):
        pltpu.make_async_copy(table_hbm.at[pl.ds(0, BW)], rows_v.at[j],
                              sem_g[j]).wait()

    def wait_writeback(s, oj):
        pltpu.make_async_copy(out_v.at[oj],
                              out_hbm.at[s].at[:, pl.ds(b0, BW)],
                              sem_w[oj]).wait()

    def compute(s, j, oj):
        rows = rows_v.at[j]
        dst = out_v.at[oj]

        for g in range(BW // L):
            tok = idx_v[s >> 3, s & 7, pl.ds(g * L, L)]
            half = (tok >> 19) << 6
            rowv = iota + g * L

            @plsc.parallel_loop(0, DIM, unroll=16)
            def _(d):
                colv = half | d
                v = plsc.load_gather(rows, [rowv, colv])
                dst.at[d][pl.ds(g * L, L)] = v

    for j in range(NBUF - 1):
        issue_gather(j, j)

    @pl.loop(0, SEQ // NBUF)
    def _(sblk):
        for j in range(NBUF):
            s = sblk * NBUF + j
            oj = j & 1

            @pl.when(s < SEQ - (NBUF - 1))
            def _():
                issue_gather(s + NBUF - 1, (j + NBUF - 1) % NBUF)

            wait_gather(j)

            @pl.when(s >= 2)
            def _():
                wait_writeback(s - 2, oj)

            compute(s, j, oj)
            pltpu.async_copy(out_v.at[oj],
                             out_hbm.at[s].at[:, pl.ds(b0, BW)],
                             sem_w[oj])

    wait_writeback(SEQ - 2, 0)
    wait_writeback(SEQ - 1, 1)


PREP_BK = 16384
SPLIT = 524288


def _prep_body(lo_ref, hi_ref, out_ref):
    out_ref[:, 0:DIM] = jnp.transpose(lo_ref[...] * SCALE)
    out_ref[:, DIM:2 * DIM] = jnp.transpose(hi_ref[...] * SCALE)

    @pl.when(pl.program_id(0) == 0)
    def _():
        out_ref[0:1, 0:DIM] = jnp.zeros((1, DIM), jnp.float32)


def _prep(table_t):
    grid = SPLIT // PREP_BK
    last = (table_t.shape[1] + PREP_BK - 1) // PREP_BK - 1
    return pl.pallas_call(
        _prep_body,
        grid=(grid,),
        in_specs=[
            pl.BlockSpec((DIM, PREP_BK), lambda i: (0, i)),
            pl.BlockSpec((DIM, PREP_BK),
                         lambda i: (0, jnp.minimum(i + grid, last))),
        ],
        out_specs=pl.BlockSpec((PREP_BK, 2 * DIM), lambda i: (i, 0)),
        out_shape=jax.ShapeDtypeStruct((SPLIT, 2 * DIM), jnp.float32),
        compiler_params=pltpu.CompilerParams(
            dimension_semantics=("parallel",)),
    )(table_t, table_t)


@jax.jit
def _embed(idx4, table2):
    batch = idx4.shape[1] * idx4.shape[3]
    mesh = plsc.VectorSubcoreMesh(core_axis_name="c", subcore_axis_name="s")
    cp = pltpu.CompilerParams(needs_layout_passes=False,
                              use_tc_tiling_on_sc=True)
    kern = pl.kernel(
        _body,
        out_type=jax.ShapeDtypeStruct((SEQ, DIM, batch), jnp.float32),
        mesh=mesh,
        scratch_types=[
            pltpu.VMEM((BANDS, 8, BW), jnp.int32),
            pltpu.VMEM((NBUF, BW), jnp.int32),
            pltpu.VMEM((NBUF, BW, 2 * DIM), jnp.float32),
            pltpu.VMEM((2, DIM, BW), jnp.float32),
            pltpu.SemaphoreType.DMA,
            pltpu.SemaphoreType.DMA,
            pltpu.SemaphoreType.DMA,
            pltpu.SemaphoreType.DMA,
            pltpu.SemaphoreType.DMA,
            pltpu.SemaphoreType.DMA,
        ],
        compiler_params=cp,
    )
    return kern(table2, idx4)


@jax.jit
def kernel(token_tensor, table):
    tt = token_tensor.astype(jnp.int32)
    batch, seq = tt.shape
    idx4 = jnp.transpose(
        jnp.transpose(tt).reshape(seq // 8, 8, batch // BW, BW),
        (0, 2, 1, 3))
    table2 = _prep(jnp.transpose(table))
    out = _embed(idx4, table2)
    return jnp.transpose(out, (2, 0, 1))
